# baseline (speedup 1.0000x reference)
import functools

import jax
import jax.numpy as jnp
from jax import lax
from jax.experimental import pallas as pl
from jax.experimental.pallas import tpu as pltpu
from jax.experimental.pallas import tpu_sc as plsc

_NC = 2
_NS = 16
_NW = _NC * _NS
_L = 16
_C = 8
_NSLOT = 8


@functools.lru_cache(maxsize=None)
def _build_embed(Bn, S, V, D):
    assert Bn == 4 and S % _NW == 0 and D % _L == 0
    SW = S // _NW
    NCH = SW // _C
    NSUP = NCH // 2
    VECS = D // _L
    BS = Bn * S
    assert NCH % 2 == 0 and NSUP >= 2

    mesh = plsc.VectorSubcoreMesh(core_axis_name="c", subcore_axis_name="s")
    scratch = (
        [pltpu.VMEM((Bn, SW), jnp.int32)]
        + [pltpu.VMEM((_C, D), jnp.float32) for _ in range(2)]
        + [pltpu.VMEM((_C, D), jnp.float32) for _ in range(_NSLOT)]
        + [pltpu.SemaphoreType.DMA for _ in range(2 + 2 * _NSLOT)]
    )

    @functools.partial(
        pl.kernel,
        mesh=mesh,
        out_type=jax.ShapeDtypeStruct((BS, D), jnp.float32),
        scratch_types=scratch,
    )
    def embed(ids_hbm, tok_hbm, pos_hbm, out_hbm, *scr):
        idx_v = scr[0]
        pos_v = scr[1:3]
        tok_v = scr[3:3 + _NSLOT]
        psem = scr[3 + _NSLOT:5 + _NSLOT]
        gsem = scr[5 + _NSLOT:5 + 2 * _NSLOT]
        osem = scr[5 + 2 * _NSLOT:5 + 3 * _NSLOT]

        wid = lax.axis_index("s") * _NC + lax.axis_index("c")
        s0 = wid * SW

        for b in range(Bn):
            pltpu.sync_copy(ids_hbm.at[b, pl.ds(s0, SW)], idx_v.at[b])

        def pos_load(c, cc):
            pltpu.async_copy(
                pos_hbm.at[pl.ds(s0 + c * _C, _C)], pos_v[cc], psem[cc])

        def pos_wait(cc):
            pltpu.make_async_copy(
                pos_hbm.at[pl.ds(0, _C)], pos_v[cc], psem[cc]).wait()

        def gather(i, p):
            cc, b = divmod(p, 4)
            c = 2 * i + cc
            pltpu.async_copy(
                tok_hbm.at[idx_v.at[b, pl.ds(c * _C, _C)]], tok_v[p], gsem[p])

        def gather_wait(p):
            pltpu.make_async_copy(
                tok_hbm.at[pl.ds(0, _C)], tok_v[p], gsem[p]).wait()

        def store(i, p):
            cc, b = divmod(p, 4)
            c = 2 * i + cc
            pltpu.async_copy(
                tok_v[p], out_hbm.at[pl.ds(b * S + s0 + c * _C, _C)], osem[p])

        def store_wait(p):
            pltpu.make_async_copy(
                tok_v[p], out_hbm.at[pl.ds(0, _C)], osem[p]).wait()

        def adds(cc, p):
            @plsc.parallel_loop(0, _C, unroll=2)
            def row(r):
                for j in range(VECS):
                    x = pos_v[cc][r, pl.ds(j * _L, _L)]
                    plsc.addupdate(tok_v[p].at[r, pl.ds(j * _L, _L)], x)

        def adds_small(cc, p):
            def row(r, carry):
                def quad(j4, carry2):
                    for j in range(VECS // 4):
                        off = j4 * (VECS // 4 * _L) + j * _L
                        x = pos_v[cc][r, pl.ds(off, _L)]
                        plsc.addupdate(tok_v[p].at[r, pl.ds(off, _L)], x)
                    return carry2
                lax.fori_loop(0, 4, quad, 0)
                return carry
            lax.fori_loop(0, _C, row, 0)

        def unit(i, p, first_super, last_super):
            cc, b = divmod(p, 4)
            peeled = first_super or last_super
            gather_wait(p)
            if p < 4:
                q = p + 4
                if not first_super:
                    store_wait(q)
                gather(i, q)
            else:
                q = p - 4
                if not last_super:
                    store_wait(q)
                    gather(i + 1, q)
            if b == 0:
                pos_wait(cc)
            (adds_small if peeled else adds)(cc, p)
            store(i, p)
            if b == 3 and not last_super:
                pos_load(2 * (i + 1) + cc, cc)

        pos_load(0, 0)
        pos_load(1, 1)
        for b in range(Bn):
            gather(0, b)
        for p in range(_NSLOT):
            unit(0, p, True, False)
        def body(i, carry):
            for p in range(_NSLOT):
                unit(i, p, False, False)
            return carry
        lax.fori_loop(1, NSUP - 1, body, 0)
        for p in range(_NSLOT):
            unit(NSUP - 1, p, False, True)
        for p in range(_NSLOT):
            store_wait(p)

    return embed


def kernel(input_ids, embed_tokens_w, embed_positions_w):
    Bn, S = input_ids.shape
    V, D = embed_tokens_w.shape
    embed = _build_embed(Bn, S, V, D)
    out = embed(input_ids, embed_tokens_w, embed_positions_w)
    return out.reshape(Bn, S, D)


# speedup vs baseline: 1.8152x; 1.8152x over previous
import functools

import jax
import jax.numpy as jnp
from jax import lax
from jax.experimental import pallas as pl
from jax.experimental.pallas import tpu as pltpu
from jax.experimental.pallas import tpu_sc as plsc

_NC = 2
_NS = 16
_NW = _NC * _NS
_L = 16
_C = 8
_NSLOT = 8


@functools.lru_cache(maxsize=None)
def _build_embed(Bn, S, V, D):
    assert Bn == 4 and S % _NW == 0 and D % _L == 0
    SW = S // _NW
    NCH = SW // _C
    NSUP = NCH // 2
    VECS = D // _L
    BS = Bn * S
    assert NCH % 2 == 0 and NSUP >= 2

    mesh = plsc.VectorSubcoreMesh(core_axis_name="c", subcore_axis_name="s")
    scratch = (
        [pltpu.VMEM((Bn, SW), jnp.int32)]
        + [pltpu.VMEM((_C, D), jnp.float32) for _ in range(2)]
        + [pltpu.VMEM((_C, D), jnp.float32) for _ in range(_NSLOT)]
        + [pltpu.SemaphoreType.DMA for _ in range(2 + 2 * _NSLOT)]
    )

    @functools.partial(
        pl.kernel,
        mesh=mesh,
        out_type=jax.ShapeDtypeStruct((BS, D), jnp.float32),
        scratch_types=scratch,
    )
    def embed(ids_hbm, tok_hbm, pos_hbm, out_hbm, *scr):
        idx_v = scr[0]
        pos_v = scr[1:3]
        tok_v = scr[3:3 + _NSLOT]
        psem = scr[3 + _NSLOT:5 + _NSLOT]
        gsem = scr[5 + _NSLOT:5 + 2 * _NSLOT]
        osem = scr[5 + 2 * _NSLOT:5 + 3 * _NSLOT]

        wid = lax.axis_index("s") * _NC + lax.axis_index("c")
        s0 = wid * SW

        for b in range(Bn):
            pltpu.async_copy(ids_hbm.at[b, pl.ds(s0, SW)], idx_v.at[b],
                             psem[0])
        for b in range(Bn):
            pltpu.make_async_copy(ids_hbm.at[b, pl.ds(s0, SW)], idx_v.at[b],
                                  psem[0]).wait()

        def pos_load(c, cc):
            pltpu.async_copy(
                pos_hbm.at[pl.ds(s0 + c * _C, _C)], pos_v[cc], psem[cc])

        def pos_wait(cc):
            pltpu.make_async_copy(
                pos_hbm.at[pl.ds(0, _C)], pos_v[cc], psem[cc]).wait()

        def gather(i, p):
            cc, b = divmod(p, 4)
            c = 2 * i + cc
            pltpu.async_copy(
                tok_hbm.at[idx_v.at[b, pl.ds(c * _C, _C)]], tok_v[p], gsem[p])

        def gather_wait(p):
            pltpu.make_async_copy(
                tok_hbm.at[pl.ds(0, _C)], tok_v[p], gsem[p]).wait()

        def store(i, p):
            cc, b = divmod(p, 4)
            c = 2 * i + cc
            pltpu.async_copy(
                tok_v[p], out_hbm.at[pl.ds(b * S + s0 + c * _C, _C)], osem[p])

        def store_wait(p):
            pltpu.make_async_copy(
                tok_v[p], out_hbm.at[pl.ds(0, _C)], osem[p]).wait()

        def adds(cc, p):
            @plsc.parallel_loop(0, _C)
            def row(r):
                for j in range(VECS):
                    x = pos_v[cc][r, pl.ds(j * _L, _L)]
                    plsc.addupdate(tok_v[p].at[r, pl.ds(j * _L, _L)], x)

        def unit(i, p, first_super, last_super):
            cc, b = divmod(p, 4)
            gather_wait(p)
            if p < 4:
                q = p + 4
                if not first_super:
                    store_wait(q)
                gather(i, q)
            else:
                q = p - 4
                if not last_super:
                    store_wait(q)
                    gather(i + 1, q)
            if b == 0:
                pos_wait(cc)
            adds(cc, p)
            store(i, p)
            if b == 3 and not last_super:
                pos_load(2 * (i + 1) + cc, cc)

        pos_load(0, 0)
        pos_load(1, 1)
        for b in range(Bn):
            gather(0, b)
        for p in range(_NSLOT):
            unit(0, p, True, False)
        def body(i, carry):
            for p in range(_NSLOT):
                unit(i, p, False, False)
            return carry
        lax.fori_loop(1, NSUP - 1, body, 0)
        for p in range(_NSLOT):
            unit(NSUP - 1, p, False, True)
        for p in range(_NSLOT):
            store_wait(p)

    return embed


def kernel(input_ids, embed_tokens_w, embed_positions_w):
    Bn, S = input_ids.shape
    V, D = embed_tokens_w.shape
    embed = _build_embed(Bn, S, V, D)
    out = embed(input_ids, embed_tokens_w, embed_positions_w)
    return out.reshape(Bn, S, D)


# speedup vs baseline: 1.8600x; 1.0247x over previous
import functools

import jax
import jax.numpy as jnp
from jax import lax
from jax.experimental import pallas as pl
from jax.experimental.pallas import tpu as pltpu
from jax.experimental.pallas import tpu_sc as plsc

_NC = 2
_NS = 16
_NW = _NC * _NS
_L = 16
_C = 8
_NSLOT = 8


@functools.lru_cache(maxsize=None)
def _build_embed(Bn, S, V, D):
    assert Bn == 4 and S % _NW == 0 and D % _L == 0
    SW = S // _NW
    NCH = SW // _C
    NSUP = NCH // 2
    VECS = D // _L
    BS = Bn * S
    assert NCH % 2 == 0 and NSUP >= 2

    mesh = plsc.VectorSubcoreMesh(core_axis_name="c", subcore_axis_name="s")
    scratch = (
        [pltpu.VMEM((Bn, SW), jnp.int32)]
        + [pltpu.VMEM((_C, D), jnp.float32) for _ in range(2)]
        + [pltpu.VMEM((_C, D), jnp.float32) for _ in range(_NSLOT)]
        + [pltpu.SemaphoreType.DMA for _ in range(2 + 2 * _NSLOT)]
    )

    @functools.partial(
        pl.kernel,
        mesh=mesh,
        out_type=jax.ShapeDtypeStruct((BS, D), jnp.float32),
        scratch_types=scratch,
    )
    def embed(ids_hbm, tok_hbm, pos_hbm, out_hbm, *scr):
        idx_v = scr[0]
        pos_v = scr[1:3]
        tok_v = scr[3:3 + _NSLOT]
        psem = scr[3 + _NSLOT:5 + _NSLOT]
        gsem = scr[5 + _NSLOT:5 + 2 * _NSLOT]
        osem = scr[5 + 2 * _NSLOT:5 + 3 * _NSLOT]

        wid = lax.axis_index("s") * _NC + lax.axis_index("c")
        s0 = wid * SW

        for b in range(Bn):
            pltpu.async_copy(ids_hbm.at[b, pl.ds(s0, SW)], idx_v.at[b],
                             psem[0])
        for b in range(Bn):
            pltpu.make_async_copy(ids_hbm.at[b, pl.ds(s0, SW)], idx_v.at[b],
                                  psem[0]).wait()

        def pos_load(c, cc):
            pltpu.async_copy(
                pos_hbm.at[pl.ds(s0 + c * _C, _C)], pos_v[cc], psem[cc])

        def pos_wait(cc):
            pltpu.make_async_copy(
                pos_hbm.at[pl.ds(0, _C)], pos_v[cc], psem[cc]).wait()

        def gather(i, p):
            cc, b = divmod(p, 4)
            c = 2 * i + cc
            pltpu.async_copy(
                tok_hbm.at[idx_v.at[b, pl.ds(c * _C, _C)]], tok_v[p], gsem[p])

        def gather_wait(p):
            pltpu.make_async_copy(
                tok_hbm.at[pl.ds(0, _C)], tok_v[p], gsem[p]).wait()

        def store(i, p):
            cc, b = divmod(p, 4)
            c = 2 * i + cc
            pltpu.async_copy(
                tok_v[p], out_hbm.at[pl.ds(b * S + s0 + c * _C, _C)], osem[p])

        def store_wait(p):
            pltpu.make_async_copy(
                tok_v[p], out_hbm.at[pl.ds(0, _C)], osem[p]).wait()

        def adds_chunk(cc):
            @plsc.parallel_loop(0, _C)
            def row(r):
                for j in range(VECS):
                    x = pos_v[cc][r, pl.ds(j * _L, _L)]
                    for b in range(Bn):
                        plsc.addupdate(
                            tok_v[cc * 4 + b].at[r, pl.ds(j * _L, _L)], x)

        def chunk_unit(i, cc, first_super, last_super):
            for b in range(Bn):
                gather_wait(cc * 4 + b)
            pos_wait(cc)
            if cc == 0:
                for b in range(Bn):
                    if not first_super:
                        store_wait(4 + b)
                    gather(i, 4 + b)
            else:
                if not last_super:
                    for b in range(Bn):
                        store_wait(b)
                        gather(i + 1, b)
            adds_chunk(cc)
            for b in range(Bn):
                store(i, cc * 4 + b)
            if not last_super:
                pos_load(2 * (i + 1) + cc, cc)

        pos_load(0, 0)
        pos_load(1, 1)
        for b in range(Bn):
            gather(0, b)
        chunk_unit(0, 0, True, False)
        chunk_unit(0, 1, True, False)
        def body(i, carry):
            chunk_unit(i, 0, False, False)
            chunk_unit(i, 1, False, False)
            return carry
        lax.fori_loop(1, NSUP - 1, body, 0)
        chunk_unit(NSUP - 1, 0, False, True)
        chunk_unit(NSUP - 1, 1, False, True)
        for p in range(_NSLOT):
            store_wait(p)

    return embed


def kernel(input_ids, embed_tokens_w, embed_positions_w):
    Bn, S = input_ids.shape
    V, D = embed_tokens_w.shape
    embed = _build_embed(Bn, S, V, D)
    out = embed(input_ids, embed_tokens_w, embed_positions_w)
    return out.reshape(Bn, S, D)
